# baseline (speedup 1.0000x reference)
import jax
import jax.numpy as jnp
from jax.experimental import pallas as pl
from jax.experimental.pallas import tpu as pltpu

M = 50
HORIZON = 30
D = 64
H = 64
N = 50000
BLK = 7168
NB = 7
RW = 896
MSEL = 64


def _ln_relu_cols(hT, gc, Bc):
    mu = jnp.mean(hT, axis=0, keepdims=True)
    dd = hT - mu
    var = jnp.mean(dd * dd, axis=0, keepdims=True)
    hn = dd / jnp.sqrt(var + 1e-5) * gc + Bc
    return jnp.maximum(hn, 0.0)


def _tnt_body(cxy_ref, cxyr_ref, W1_ref, W2_ref, V_ref,
              trajsT_out_ref, score_out_ref, Ls):
    V = V_ref[...]
    featT = V[:, 0:1]
    featB = jnp.broadcast_to(featT, (D, BLK))
    neg = jnp.float32(-jnp.inf)

    tp_W1T = W1_ref[:, 0:66]
    tp_w2r = W2_ref[0:1, :]
    tp_b1c, tp_g, tp_B = V[:, 1:2], V[:, 2:3], V[:, 3:4]
    tp_b2 = V[0:1, 15:16]

    cms = []
    for j in range(NB):
        cxr = cxy_ref[j:j + 1, :]
        cyr = cxy_ref[NB + j:NB + j + 1, :]
        xbT = jnp.concatenate([featB, cxr, cyr], axis=0)
        hT = jnp.dot(tp_W1T, xbT,
                     preferred_element_type=jnp.float32) + tp_b1c
        hr = _ln_relu_cols(hT, tp_g, tp_B)
        lg = jnp.dot(tp_w2r, hr,
                     preferred_element_type=jnp.float32) + tp_b2
        if (j + 1) * BLK > N:
            lane = jax.lax.broadcasted_iota(jnp.int32, (1, BLK), 1)
            lg = jnp.where(j * BLK + lane < N, lg, neg)
        for t in range(BLK // RW):
            Ls[8 * j + t:8 * j + t + 1, :] = lg[:, t * RW:(t + 1) * RW]
        cms.append(jnp.max(lg, axis=(0, 1), keepdims=True))

    BIG = jnp.int32(2 ** 30)
    lane64 = jax.lax.broadcasted_iota(jnp.int32, (1, MSEL), 1)
    iota128 = jax.lax.broadcasted_iota(jnp.int32, (1, 128), 1)
    lflat = (jax.lax.broadcasted_iota(jnp.int32, (8, RW), 0) * RW
             + jax.lax.broadcasted_iota(jnp.int32, (8, RW), 1))
    CM0 = jnp.concatenate(
        cms + [jnp.full((1, 128 - NB), neg, jnp.float32)], axis=1)

    def sel_body(jj, carry):
        CM, sxT, syT = carry
        mv = jnp.max(CM, axis=(0, 1), keepdims=True)
        qv = jnp.min(jnp.where(CM == mv, iota128, BIG),
                     axis=(0, 1), keepdims=True)
        q = qv[0, 0]
        C = Ls[pl.ds(q * 8, 8), :]
        pickl = jnp.min(jnp.where(C == mv, lflat, BIG),
                        axis=(0, 1), keepdims=True)
        hit = lflat == pickl
        Cx = cxyr_ref[pl.ds(q * 8, 8), :]
        Cy = cxyr_ref[pl.ds(56 + q * 8, 8), :]
        x = jnp.sum(jnp.where(hit, Cx, 0.0), axis=(0, 1), keepdims=True)
        y = jnp.sum(jnp.where(hit, Cy, 0.0), axis=(0, 1), keepdims=True)
        Cn = jnp.where(hit, neg, C)
        Ls[pl.ds(q * 8, 8), :] = Cn
        cmn = jnp.max(Cn, axis=(0, 1), keepdims=True)
        CM = jnp.where(iota128 == qv, cmn, CM)
        sxT = jnp.where(lane64 == jj, x, sxT)
        syT = jnp.where(lane64 == jj, y, syT)
        return CM, sxT, syT

    z = jnp.zeros((1, MSEL), jnp.float32)
    _, sxT, syT = jax.lax.fori_loop(0, M, sel_body, (CM0, z, z))

    featB64 = jnp.broadcast_to(featT, (D, MSEL))

    xselT = jnp.concatenate([featB64, sxT, syT], axis=0)
    h2 = jnp.dot(W1_ref[:, 128:194], xselT,
                 preferred_element_type=jnp.float32) + V[:, 4:5]
    hr2 = _ln_relu_cols(h2, V[:, 5:6], V[:, 6:7])
    offT = jnp.dot(W2_ref[1:3, :], hr2,
                   preferred_element_type=jnp.float32) + V[0:2, 13:14]
    locT = jnp.concatenate([sxT, syT], axis=0) + offT

    xinT = jnp.concatenate([featB64, locT], axis=0)
    h3 = jnp.dot(W1_ref[:, 256:322], xinT,
                 preferred_element_type=jnp.float32) + V[:, 7:8]
    hr3 = _ln_relu_cols(h3, V[:, 8:9], V[:, 9:10])
    trajsT = jnp.dot(W2_ref[3:63, :], hr3,
                     preferred_element_type=jnp.float32) + V[0:60, 14:15]

    xsT = jnp.concatenate([featB64, trajsT], axis=0)
    h4 = jnp.dot(W1_ref[:, 384:508], xsT,
                 preferred_element_type=jnp.float32) + V[:, 10:11]
    hr4 = _ln_relu_cols(h4, V[:, 11:12], V[:, 12:13])
    slog = jnp.dot(W2_ref[63:64, :], hr4,
                   preferred_element_type=jnp.float32) + V[1:2, 15:16]
    validc = lane64 < M
    slog = jnp.where(validc, slog, neg)
    sm = jnp.max(slog)
    e = jnp.where(validc, jnp.exp(slog - sm), 0.0)
    score = e / jnp.sum(e)

    trajsT_out_ref[...] = trajsT[:, :M]
    score_out_ref[...] = score[:, :M]


def kernel(target_feat, target_candidate, tp_W1, tp_b1, tp_g, tp_B, tp_W2, tp_b2,
           tm_W1, tm_b1, tm_g, tm_B, tm_W2, tm_b2,
           me_W1, me_b1, me_g, me_B, me_W2, me_b2,
           ts_W1, ts_b1, ts_g, ts_B, ts_W2, ts_b2):
    c = jnp.pad(target_candidate, ((0, NB * BLK - N), (0, 0)))
    cxy = jnp.concatenate(
        [c[:, 0].reshape(NB, BLK), c[:, 1].reshape(NB, BLK)], axis=0)
    cxyr = jnp.concatenate(
        [c[:, 0].reshape(56, RW), c[:, 1].reshape(56, RW)], axis=0)

    z1 = jnp.zeros((D, 128), jnp.float32)
    W1pack = jnp.concatenate([
        jnp.pad(tp_W1.T, ((0, 0), (0, 62))),
        jnp.pad(tm_W1.T, ((0, 0), (0, 62))),
        jnp.pad(me_W1.T, ((0, 0), (0, 62))),
        jnp.pad(ts_W1.T, ((0, 0), (0, 4))),
    ], axis=1)
    del z1

    W2pack = jnp.concatenate([tp_W2.T, tm_W2.T, me_W2.T, ts_W2.T], axis=0)

    def colp(v):
        v = v.reshape(-1, 1)
        return jnp.pad(v, ((0, D - v.shape[0]), (0, 0)))

    V = jnp.concatenate([
        colp(target_feat), colp(tp_b1), colp(tp_g), colp(tp_B),
        colp(tm_b1), colp(tm_g), colp(tm_B),
        colp(me_b1), colp(me_g), colp(me_B),
        colp(ts_b1), colp(ts_g), colp(ts_B),
        colp(tm_b2), colp(me_b2),
        colp(jnp.concatenate([tp_b2, ts_b2])),
    ], axis=1)

    full = lambda i: (0, 0)
    args = (cxy, cxyr, W1pack, W2pack, V)
    trajsT, score = pl.pallas_call(
        _tnt_body,
        grid=(1,),
        in_specs=[pl.BlockSpec(a.shape, full) for a in args],
        out_specs=(
            pl.BlockSpec((HORIZON * 2, M), full),
            pl.BlockSpec((1, M), full),
        ),
        out_shape=(
            jax.ShapeDtypeStruct((HORIZON * 2, M), jnp.float32),
            jax.ShapeDtypeStruct((1, M), jnp.float32),
        ),
        scratch_shapes=[pltpu.VMEM((56, RW), jnp.float32)],
    )(*args)
    return trajsT.T, score.reshape(M)


# speedup vs baseline: 1.0453x; 1.0453x over previous
import jax
import jax.numpy as jnp
from jax.experimental import pallas as pl

M = 50
HORIZON = 30
D = 64
H = 64
N = 50000
BLK = 7168
NB = 7
MSEL = 64
W2OFF = 512
VOFF = 576


def _ln_relu_cols(hT, gc, Bc):
    mu = jnp.mean(hT, axis=0, keepdims=True)
    dd = hT - mu
    var = jnp.mean(dd * dd, axis=0, keepdims=True)
    hn = dd / jnp.sqrt(var + 1e-5) * gc + Bc
    return jnp.maximum(hn, 0.0)


def _tnt_body(cxy_ref, WP_ref, out_ref):
    V = WP_ref[:, VOFF:VOFF + 16]
    featT = V[:, 0:1]
    featB = jnp.broadcast_to(featT, (D, BLK))
    neg = jnp.float32(-jnp.inf)

    tp_W1T = WP_ref[:, 0:66]
    tp_w2r = WP_ref[0:1, W2OFF:W2OFF + 64]
    tp_b1c, tp_g, tp_B = V[:, 1:2], V[:, 2:3], V[:, 3:4]
    tp_b2 = V[0:1, 15:16]

    rows = []
    for j in range(NB):
        cxr = cxy_ref[j:j + 1, :]
        cyr = cxy_ref[NB + j:NB + j + 1, :]
        xbT = jnp.concatenate([featB, cxr, cyr], axis=0)
        hT = jnp.dot(tp_W1T, xbT,
                     preferred_element_type=jnp.float32) + tp_b1c
        hr = _ln_relu_cols(hT, tp_g, tp_B)
        lg = jnp.dot(tp_w2r, hr,
                     preferred_element_type=jnp.float32) + tp_b2
        if (j + 1) * BLK > N:
            lane = jax.lax.broadcasted_iota(jnp.int32, (1, BLK), 1)
            lg = jnp.where(j * BLK + lane < N, lg, neg)
        rows.append(lg)
    L0 = jnp.concatenate(rows, axis=0)

    cx2 = cxy_ref[0:NB, :]
    cy2 = cxy_ref[NB:2 * NB, :]
    flat2 = (jax.lax.broadcasted_iota(jnp.int32, (NB, BLK), 0) * BLK
             + jax.lax.broadcasted_iota(jnp.int32, (NB, BLK), 1))
    BIG = jnp.int32(2 ** 30)
    lane64 = jax.lax.broadcasted_iota(jnp.int32, (1, MSEL), 1)

    L = L0
    xs, ys = [], []
    for _ in range(M):
        mv = jnp.max(L, axis=(0, 1), keepdims=True)
        pickv = jnp.min(jnp.where(L == mv, flat2, BIG),
                        axis=(0, 1), keepdims=True)
        hit = flat2 == pickv
        xs.append(jnp.sum(jnp.where(hit, cx2, 0.0),
                          axis=(0, 1), keepdims=True))
        ys.append(jnp.sum(jnp.where(hit, cy2, 0.0),
                          axis=(0, 1), keepdims=True))
        L = jnp.where(hit, neg, L)
    pad14 = jnp.zeros((1, MSEL - M), jnp.float32)
    sxT = jnp.concatenate(xs + [pad14], axis=1)
    syT = jnp.concatenate(ys + [pad14], axis=1)

    featB64 = jnp.broadcast_to(featT, (D, MSEL))

    xselT = jnp.concatenate([featB64, sxT, syT], axis=0)
    h2 = jnp.dot(WP_ref[:, 128:194], xselT,
                 preferred_element_type=jnp.float32) + V[:, 4:5]
    hr2 = _ln_relu_cols(h2, V[:, 5:6], V[:, 6:7])
    offT = jnp.dot(WP_ref[1:3, W2OFF:W2OFF + 64], hr2,
                   preferred_element_type=jnp.float32) + V[0:2, 13:14]
    locT = jnp.concatenate([sxT, syT], axis=0) + offT

    xinT = jnp.concatenate([featB64, locT], axis=0)
    h3 = jnp.dot(WP_ref[:, 256:322], xinT,
                 preferred_element_type=jnp.float32) + V[:, 7:8]
    hr3 = _ln_relu_cols(h3, V[:, 8:9], V[:, 9:10])
    trajsT = jnp.dot(WP_ref[3:63, W2OFF:W2OFF + 64], hr3,
                     preferred_element_type=jnp.float32) + V[0:60, 14:15]

    xsT = jnp.concatenate([featB64, trajsT], axis=0)
    h4 = jnp.dot(WP_ref[:, 384:508], xsT,
                 preferred_element_type=jnp.float32) + V[:, 10:11]
    hr4 = _ln_relu_cols(h4, V[:, 11:12], V[:, 12:13])
    slog = jnp.dot(WP_ref[63:64, W2OFF:W2OFF + 64], hr4,
                   preferred_element_type=jnp.float32) + V[1:2, 15:16]
    validc = lane64 < M
    slog = jnp.where(validc, slog, neg)
    sm = jnp.max(slog)
    e = jnp.where(validc, jnp.exp(slog - sm), 0.0)
    score = e / jnp.sum(e)

    out_ref[...] = jnp.concatenate([trajsT, score], axis=0)


def kernel(target_feat, target_candidate, tp_W1, tp_b1, tp_g, tp_B, tp_W2, tp_b2,
           tm_W1, tm_b1, tm_g, tm_B, tm_W2, tm_b2,
           me_W1, me_b1, me_g, me_B, me_W2, me_b2,
           ts_W1, ts_b1, ts_g, ts_B, ts_W2, ts_b2):
    c = jnp.pad(target_candidate, ((0, NB * BLK - N), (0, 0)))
    cxy = jnp.concatenate(
        [c[:, 0].reshape(NB, BLK), c[:, 1].reshape(NB, BLK)], axis=0)

    def colp(v):
        v = v.reshape(-1, 1)
        return jnp.pad(v, ((0, D - v.shape[0]), (0, 0)))

    WP = jnp.concatenate([
        jnp.pad(tp_W1.T, ((0, 0), (0, 62))),
        jnp.pad(tm_W1.T, ((0, 0), (0, 62))),
        jnp.pad(me_W1.T, ((0, 0), (0, 62))),
        jnp.pad(ts_W1.T, ((0, 0), (0, 4))),
        jnp.concatenate([tp_W2.T, tm_W2.T, me_W2.T, ts_W2.T], axis=0),
        colp(target_feat), colp(tp_b1), colp(tp_g), colp(tp_B),
        colp(tm_b1), colp(tm_g), colp(tm_B),
        colp(me_b1), colp(me_g), colp(me_B),
        colp(ts_b1), colp(ts_g), colp(ts_B),
        colp(tm_b2), colp(me_b2),
        colp(jnp.concatenate([tp_b2, ts_b2])),
    ], axis=1)

    full = lambda i: (0, 0)
    args = (cxy, WP)
    out = pl.pallas_call(
        _tnt_body,
        grid=(1,),
        in_specs=[pl.BlockSpec(a.shape, full) for a in args],
        out_specs=pl.BlockSpec((HORIZON * 2 + 1, MSEL), full),
        out_shape=jax.ShapeDtypeStruct((HORIZON * 2 + 1, MSEL), jnp.float32),
    )(*args)
    return out[:HORIZON * 2, :M].T, out[HORIZON * 2, :M]


# speedup vs baseline: 1.1662x; 1.1157x over previous
import jax
import jax.numpy as jnp
from jax.experimental import pallas as pl

M = 50
HORIZON = 30
D = 64
H = 64
N = 50000
BLK = 7168
NB = 7
MSEL = 64
W2OFF = 512
VOFF = 576


def _ln_relu_cols(hT, gc, Bc):
    mu = jnp.mean(hT, axis=0, keepdims=True)
    dd = hT - mu
    var = jnp.mean(dd * dd, axis=0, keepdims=True)
    hn = dd / jnp.sqrt(var + 1e-5) * gc + Bc
    return jnp.maximum(hn, 0.0)


def _tnt_body(cxy_ref, WP_ref, V_ref, out_ref):
    V = V_ref[...]
    featT = V[:, 0:1]
    featB = jnp.broadcast_to(featT, (D, BLK))
    neg = jnp.float32(-jnp.inf)

    tp_W1T = WP_ref[:, 0:66]
    tp_w2r = WP_ref[0:1, W2OFF:W2OFF + 64]
    tp_b1c, tp_g, tp_B = V[:, 1:2], V[:, 2:3], V[:, 3:4]
    tp_b2 = V[0:1, 15:16]

    rows = []
    for j in range(NB):
        cxr = cxy_ref[j:j + 1, :]
        cyr = cxy_ref[NB + j:NB + j + 1, :]
        xbT = jnp.concatenate([featB, cxr, cyr], axis=0)
        hT = jnp.dot(tp_W1T, xbT,
                     preferred_element_type=jnp.float32) + tp_b1c
        hr = _ln_relu_cols(hT, tp_g, tp_B)
        lg = jnp.dot(tp_w2r, hr,
                     preferred_element_type=jnp.float32) + tp_b2
        if (j + 1) * BLK > N:
            lane = jax.lax.broadcasted_iota(jnp.int32, (1, BLK), 1)
            lg = jnp.where(j * BLK + lane < N, lg, neg)
        rows.append(lg)
    L0 = jnp.concatenate(rows, axis=0)

    cx2 = cxy_ref[0:NB, :]
    cy2 = cxy_ref[NB:2 * NB, :]
    flat2 = (jax.lax.broadcasted_iota(jnp.int32, (NB, BLK), 0) * BLK
             + jax.lax.broadcasted_iota(jnp.int32, (NB, BLK), 1))
    BIG = jnp.int32(2 ** 30)
    lane64 = jax.lax.broadcasted_iota(jnp.int32, (1, MSEL), 1)

    L = L0
    xs, ys = [], []
    for _ in range(M):
        mv = jnp.max(L, axis=(0, 1), keepdims=True)
        pickv = jnp.min(jnp.where(L == mv, flat2, BIG),
                        axis=(0, 1), keepdims=True)
        hit = flat2 == pickv
        xs.append(jnp.sum(jnp.where(hit, cx2, 0.0),
                          axis=(0, 1), keepdims=True))
        ys.append(jnp.sum(jnp.where(hit, cy2, 0.0),
                          axis=(0, 1), keepdims=True))
        L = jnp.where(hit, neg, L)
    pad14 = jnp.zeros((1, MSEL - M), jnp.float32)
    sxT = jnp.concatenate(xs + [pad14], axis=1)
    syT = jnp.concatenate(ys + [pad14], axis=1)

    featB64 = jnp.broadcast_to(featT, (D, MSEL))

    xselT = jnp.concatenate([featB64, sxT, syT], axis=0)
    h2 = jnp.dot(WP_ref[:, 128:194], xselT,
                 preferred_element_type=jnp.float32) + V[:, 4:5]
    hr2 = _ln_relu_cols(h2, V[:, 5:6], V[:, 6:7])
    offT = jnp.dot(WP_ref[1:3, W2OFF:W2OFF + 64], hr2,
                   preferred_element_type=jnp.float32) + V[0:2, 13:14]
    locT = jnp.concatenate([sxT, syT], axis=0) + offT

    xinT = jnp.concatenate([featB64, locT], axis=0)
    h3 = jnp.dot(WP_ref[:, 256:322], xinT,
                 preferred_element_type=jnp.float32) + V[:, 7:8]
    hr3 = _ln_relu_cols(h3, V[:, 8:9], V[:, 9:10])
    trajsT = jnp.dot(WP_ref[3:63, W2OFF:W2OFF + 64], hr3,
                     preferred_element_type=jnp.float32) + V[0:60, 14:15]

    xsT = jnp.concatenate([featB64, trajsT], axis=0)
    h4 = jnp.dot(WP_ref[:, 384:508], xsT,
                 preferred_element_type=jnp.float32) + V[:, 10:11]
    hr4 = _ln_relu_cols(h4, V[:, 11:12], V[:, 12:13])
    slog = jnp.dot(WP_ref[63:64, W2OFF:W2OFF + 64], hr4,
                   preferred_element_type=jnp.float32) + V[1:2, 15:16]
    validc = lane64 < M
    slog = jnp.where(validc, slog, neg)
    sm = jnp.max(slog)
    e = jnp.where(validc, jnp.exp(slog - sm), 0.0)
    score = e / jnp.sum(e)

    out_ref[...] = jnp.concatenate([trajsT, score], axis=0)


def kernel(target_feat, target_candidate, tp_W1, tp_b1, tp_g, tp_B, tp_W2, tp_b2,
           tm_W1, tm_b1, tm_g, tm_B, tm_W2, tm_b2,
           me_W1, me_b1, me_g, me_B, me_W2, me_b2,
           ts_W1, ts_b1, ts_g, ts_B, ts_W2, ts_b2):
    c = jnp.pad(target_candidate, ((0, NB * BLK - N), (0, 0)))
    cxy = jnp.concatenate(
        [c[:, 0].reshape(NB, BLK), c[:, 1].reshape(NB, BLK)], axis=0)

    def colp(v):
        v = v.reshape(-1, 1)
        return jnp.pad(v, ((0, D - v.shape[0]), (0, 0)))

    WP = jnp.concatenate([
        jnp.pad(tp_W1.T, ((0, 0), (0, 62))),
        jnp.pad(tm_W1.T, ((0, 0), (0, 62))),
        jnp.pad(me_W1.T, ((0, 0), (0, 62))),
        jnp.pad(ts_W1.T, ((0, 0), (0, 4))),
        jnp.concatenate([tp_W2.T, tm_W2.T, me_W2.T, ts_W2.T], axis=0),
    ], axis=1)

    V = jnp.concatenate([
        colp(target_feat), colp(tp_b1), colp(tp_g), colp(tp_B),
        colp(tm_b1), colp(tm_g), colp(tm_B),
        colp(me_b1), colp(me_g), colp(me_B),
        colp(ts_b1), colp(ts_g), colp(ts_B),
        colp(tm_b2), colp(me_b2),
        colp(jnp.concatenate([tp_b2, ts_b2])),
    ], axis=1)

    full = lambda i: (0, 0)
    args = (cxy, WP, V)
    out = pl.pallas_call(
        _tnt_body,
        grid=(1,),
        in_specs=[pl.BlockSpec(a.shape, full) for a in args],
        out_specs=pl.BlockSpec((HORIZON * 2 + 1, MSEL), full),
        out_shape=jax.ShapeDtypeStruct((HORIZON * 2 + 1, MSEL), jnp.float32),
    )(*args)
    return out[:HORIZON * 2, :M].T, out[HORIZON * 2, :M]


# speedup vs baseline: 1.2533x; 1.0747x over previous
import jax
import jax.numpy as jnp
from jax.experimental import pallas as pl

M = 50
HORIZON = 30
D = 64
H = 64
N = 50000
BLK = 7168
NB = 7
MSEL = 64
W2OFF = 512
VOFF = 576


def _ln_relu_cols(hT, gc, Bc):
    mu = jnp.mean(hT, axis=0, keepdims=True)
    dd = hT - mu
    var = jnp.mean(dd * dd, axis=0, keepdims=True)
    hn = dd / jnp.sqrt(var + 1e-5) * gc + Bc
    return jnp.maximum(hn, 0.0)


def _tnt_body(cxy_ref, WP_ref, V_ref, out_ref):
    V = V_ref[...]
    featT = V[:, 0:1]
    featB = jnp.broadcast_to(featT, (D, BLK))
    neg = jnp.float32(-jnp.inf)

    tp_W1T = WP_ref[:, 0:66]
    tp_w2r = WP_ref[0:1, W2OFF:W2OFF + 64]
    tp_b1c, tp_g, tp_B = V[:, 1:2], V[:, 2:3], V[:, 3:4]
    tp_b2 = V[0:1, 15:16]

    rows = []
    for j in range(NB):
        cxr = cxy_ref[j:j + 1, :]
        cyr = cxy_ref[NB + j:NB + j + 1, :]
        xbT = jnp.concatenate([featB, cxr, cyr], axis=0)
        hT = jnp.dot(tp_W1T, xbT,
                     preferred_element_type=jnp.float32) + tp_b1c
        hr = _ln_relu_cols(hT, tp_g, tp_B)
        lg = jnp.dot(tp_w2r, hr,
                     preferred_element_type=jnp.float32) + tp_b2
        if (j + 1) * BLK > N:
            lane = jax.lax.broadcasted_iota(jnp.int32, (1, BLK), 1)
            lg = jnp.where(j * BLK + lane < N, lg, neg)
        rows.append(lg)
    L0 = jnp.concatenate(rows, axis=0)

    cx2 = cxy_ref[0:NB, :]
    cy2 = cxy_ref[NB:2 * NB, :]
    flat2 = (jax.lax.broadcasted_iota(jnp.int32, (NB, BLK), 0) * BLK
             + jax.lax.broadcasted_iota(jnp.int32, (NB, BLK), 1))
    BIG = jnp.int32(2 ** 30)
    lane64 = jax.lax.broadcasted_iota(jnp.int32, (1, MSEL), 1)

    L = L0
    xs, ys = [], []
    for _ in range(M // 2):
        m1 = jnp.max(L, axis=(0, 1), keepdims=True)
        eq1 = L == m1
        pick1 = jnp.min(jnp.where(eq1, flat2, BIG),
                        axis=(0, 1), keepdims=True)
        m2x = jnp.max(jnp.where(eq1, neg, L),
                      axis=(0, 1), keepdims=True)
        cnt = jnp.sum(eq1.astype(jnp.int32), axis=(0, 1), keepdims=True)
        m2 = jnp.where(cnt >= 2, m1, m2x)
        pick2 = jnp.min(jnp.where((L == m2) & (flat2 != pick1), flat2, BIG),
                        axis=(0, 1), keepdims=True)
        hit1 = flat2 == pick1
        hit2 = flat2 == pick2
        xs.append(jnp.sum(jnp.where(hit1, cx2, 0.0),
                          axis=(0, 1), keepdims=True))
        ys.append(jnp.sum(jnp.where(hit1, cy2, 0.0),
                          axis=(0, 1), keepdims=True))
        xs.append(jnp.sum(jnp.where(hit2, cx2, 0.0),
                          axis=(0, 1), keepdims=True))
        ys.append(jnp.sum(jnp.where(hit2, cy2, 0.0),
                          axis=(0, 1), keepdims=True))
        L = jnp.where(hit1 | hit2, neg, L)
    pad14 = jnp.zeros((1, MSEL - M), jnp.float32)
    sxT = jnp.concatenate(xs + [pad14], axis=1)
    syT = jnp.concatenate(ys + [pad14], axis=1)

    featB64 = jnp.broadcast_to(featT, (D, MSEL))

    xselT = jnp.concatenate([featB64, sxT, syT], axis=0)
    h2 = jnp.dot(WP_ref[:, 128:194], xselT,
                 preferred_element_type=jnp.float32) + V[:, 4:5]
    hr2 = _ln_relu_cols(h2, V[:, 5:6], V[:, 6:7])
    offT = jnp.dot(WP_ref[1:3, W2OFF:W2OFF + 64], hr2,
                   preferred_element_type=jnp.float32) + V[0:2, 13:14]
    locT = jnp.concatenate([sxT, syT], axis=0) + offT

    xinT = jnp.concatenate([featB64, locT], axis=0)
    h3 = jnp.dot(WP_ref[:, 256:322], xinT,
                 preferred_element_type=jnp.float32) + V[:, 7:8]
    hr3 = _ln_relu_cols(h3, V[:, 8:9], V[:, 9:10])
    trajsT = jnp.dot(WP_ref[3:63, W2OFF:W2OFF + 64], hr3,
                     preferred_element_type=jnp.float32) + V[0:60, 14:15]

    xsT = jnp.concatenate([featB64, trajsT], axis=0)
    h4 = jnp.dot(WP_ref[:, 384:508], xsT,
                 preferred_element_type=jnp.float32) + V[:, 10:11]
    hr4 = _ln_relu_cols(h4, V[:, 11:12], V[:, 12:13])
    slog = jnp.dot(WP_ref[63:64, W2OFF:W2OFF + 64], hr4,
                   preferred_element_type=jnp.float32) + V[1:2, 15:16]
    validc = lane64 < M
    slog = jnp.where(validc, slog, neg)
    sm = jnp.max(slog)
    e = jnp.where(validc, jnp.exp(slog - sm), 0.0)
    score = e / jnp.sum(e)

    out_ref[...] = jnp.concatenate([trajsT, score], axis=0)


def kernel(target_feat, target_candidate, tp_W1, tp_b1, tp_g, tp_B, tp_W2, tp_b2,
           tm_W1, tm_b1, tm_g, tm_B, tm_W2, tm_b2,
           me_W1, me_b1, me_g, me_B, me_W2, me_b2,
           ts_W1, ts_b1, ts_g, ts_B, ts_W2, ts_b2):
    c = jnp.pad(target_candidate, ((0, NB * BLK - N), (0, 0)))
    cxy = jnp.concatenate(
        [c[:, 0].reshape(NB, BLK), c[:, 1].reshape(NB, BLK)], axis=0)

    def colp(v):
        v = v.reshape(-1, 1)
        return jnp.pad(v, ((0, D - v.shape[0]), (0, 0)))

    WP = jnp.concatenate([
        jnp.pad(tp_W1.T, ((0, 0), (0, 62))),
        jnp.pad(tm_W1.T, ((0, 0), (0, 62))),
        jnp.pad(me_W1.T, ((0, 0), (0, 62))),
        jnp.pad(ts_W1.T, ((0, 0), (0, 4))),
        jnp.concatenate([tp_W2.T, tm_W2.T, me_W2.T, ts_W2.T], axis=0),
    ], axis=1)

    V = jnp.concatenate([
        colp(target_feat), colp(tp_b1), colp(tp_g), colp(tp_B),
        colp(tm_b1), colp(tm_g), colp(tm_B),
        colp(me_b1), colp(me_g), colp(me_B),
        colp(ts_b1), colp(ts_g), colp(ts_B),
        colp(tm_b2), colp(me_b2),
        colp(jnp.concatenate([tp_b2, ts_b2])),
    ], axis=1)

    full = lambda i: (0, 0)
    args = (cxy, WP, V)
    out = pl.pallas_call(
        _tnt_body,
        grid=(1,),
        in_specs=[pl.BlockSpec(a.shape, full) for a in args],
        out_specs=pl.BlockSpec((HORIZON * 2 + 1, MSEL), full),
        out_shape=jax.ShapeDtypeStruct((HORIZON * 2 + 1, MSEL), jnp.float32),
    )(*args)
    return out[:HORIZON * 2, :M].T, out[HORIZON * 2, :M]
